# baseline (speedup 1.0000x reference)
import functools

import jax
import jax.numpy as jnp
from jax import lax
from jax.experimental import pallas as pl
from jax.experimental.pallas import tpu as pltpu
from jax.experimental.pallas import tpu_sc as plsc

N, D, HW = 8, 32, 1024
C_IMG = 3
K = 8192
KT = 512
NKT = K // KT
P = N * HW
COMMITMENT_WEIGHT = 0.25
VQLOSS_WEIGHT = 1.0

SC_NC, SC_NS = 2, 16
SC_NW = SC_NC * SC_NS
ROWS_PER_W = P // SC_NW


def _argmin_body(lat_ref, cb_ref, idx_ref, bv_ref, bi_ref):
    k = pl.program_id(1)
    latn = lat_ref[0]
    cb = cb_ref[...]
    c2 = jnp.sum(cb * cb, axis=1, keepdims=True)
    xc = lax.dot_general(cb, latn, (((1,), (0,)), ((), ())),
                         preferred_element_type=jnp.float32,
                         precision=lax.Precision.HIGHEST)
    s = xc - 0.5 * c2
    m = jnp.max(s, axis=0, keepdims=True)
    eq = (s == m).astype(jnp.float32)
    iota = lax.broadcasted_iota(jnp.int32, (1, KT), 1).astype(jnp.float32)
    loc = lax.dot_general(iota, eq, (((1,), (0,)), ((), ())),
                          preferred_element_type=jnp.float32,
                          precision=lax.Precision.HIGHEST)

    @pl.when(k == 0)
    def _init():
        bv_ref[...] = jnp.full((1, HW), -jnp.inf, jnp.float32)
        bi_ref[...] = jnp.zeros((1, HW), jnp.float32)

    upd = m > bv_ref[...]
    bi_ref[...] = jnp.where(upd, loc + jnp.float32(k * KT), bi_ref[...])
    bv_ref[...] = jnp.where(upd, m, bv_ref[...])

    @pl.when(k == NKT - 1)
    def _emit():
        idx_ref[0] = jnp.clip(bi_ref[...], 0.0, float(K - 1)).astype(jnp.int32)


def _nearest_indices(lat3, codebook, interpret=False):
    return pl.pallas_call(
        _argmin_body,
        grid=(N, NKT),
        in_specs=[
            pl.BlockSpec((1, D, HW), lambda n, k: (n, 0, 0)),
            pl.BlockSpec((KT, D), lambda n, k: (k, 0)),
        ],
        out_specs=pl.BlockSpec((1, 1, HW), lambda n, k: (n, 0, 0)),
        out_shape=jax.ShapeDtypeStruct((N, 1, HW), jnp.int32),
        scratch_shapes=[
            pltpu.VMEM((1, HW), jnp.float32),
            pltpu.VMEM((1, HW), jnp.float32),
        ],
        interpret=interpret,
    )(lat3, codebook)


def _sc_gather(idx_flat, codebook):
    mesh = plsc.VectorSubcoreMesh(core_axis_name="c", subcore_axis_name="s")

    @functools.partial(
        pl.kernel,
        mesh=mesh,
        compiler_params=pltpu.CompilerParams(use_tc_tiling_on_sc=False),
        out_type=jax.ShapeDtypeStruct((P, D), jnp.float32),
        scratch_types=[
            pltpu.VMEM((ROWS_PER_W,), jnp.int32),
            pltpu.VMEM((ROWS_PER_W, D), jnp.float32),
            pltpu.SemaphoreType.DMA,
        ],
    )
    def gather_kernel(idx_hbm, cb_hbm, out_hbm, idx_v, rows_v, sem):
        wid = lax.axis_index("s") * SC_NC + lax.axis_index("c")
        base = wid * ROWS_PER_W
        pltpu.sync_copy(idx_hbm.at[pl.ds(base, ROWS_PER_W)], idx_v)
        pltpu.async_copy(cb_hbm.at[idx_v], rows_v, sem).wait()
        pltpu.sync_copy(rows_v, out_hbm.at[pl.ds(base, ROWS_PER_W)])

    return gather_kernel(idx_flat, codebook)


def _loss_body(lat_ref, q_ref, tgt_ref, w_ref, b_ref,
               loss_ref, vq_ref, rec_ref):
    eye = (lax.broadcasted_iota(jnp.int32, (D, D), 0)
           == lax.broadcasted_iota(jnp.int32, (D, D), 1))
    w = w_ref[...]
    b = b_ref[...]
    cb_sum = jnp.float32(0.0)
    rec_sum = jnp.float32(0.0)
    for n in range(N):
        latn = lat_ref[n]
        qn = q_ref[pl.ds(n * HW, HW), :]
        m = lax.dot_general(latn, qn, (((1,), (0,)), ((), ())),
                            preferred_element_type=jnp.float32,
                            precision=lax.Precision.HIGHEST)
        cross = jnp.sum(jnp.where(eye, m, 0.0))
        cb_sum = cb_sum + jnp.sum(latn * latn) + jnp.sum(qn * qn) - 2.0 * cross
        out = lax.dot_general(w, qn, (((1,), (1,)), ((), ())),
                              preferred_element_type=jnp.float32,
                              precision=lax.Precision.HIGHEST) + b
        r = out - tgt_ref[n]
        rec_sum = rec_sum + jnp.sum(r * r)
    cb_mean = cb_sum / jnp.float32(P * D)
    vq = (VQLOSS_WEIGHT + VQLOSS_WEIGHT * COMMITMENT_WEIGHT) * cb_mean
    rec = rec_sum / jnp.float32(P * C_IMG)
    loss_ref[0, 0] = vq + rec
    vq_ref[0, 0] = vq
    rec_ref[0, 0] = rec


def _losses(lat3, q, tgt3, w_dec, b_dec2, interpret=False):
    smem_out = pl.BlockSpec(memory_space=pltpu.SMEM)
    return pl.pallas_call(
        _loss_body,
        in_specs=[pl.BlockSpec(memory_space=pltpu.VMEM)] * 5,
        out_specs=(smem_out, smem_out, smem_out),
        out_shape=(
            jax.ShapeDtypeStruct((1, 1), jnp.float32),
            jax.ShapeDtypeStruct((1, 1), jnp.float32),
            jax.ShapeDtypeStruct((1, 1), jnp.float32),
        ),
        interpret=interpret,
    )(lat3, q, tgt3, w_dec, b_dec2)


def kernel(latents, target, codebook, W_dec, b_dec):
    lat3 = latents.reshape(N, D, HW)
    tgt3 = target.reshape(N, C_IMG, HW)
    idx = _nearest_indices(lat3, codebook)
    q = _sc_gather(idx.reshape(P), codebook)
    loss, vq, rec = _losses(lat3, q, tgt3, W_dec, b_dec.reshape(C_IMG, 1))
    return (loss.reshape(()), vq.reshape(()), rec.reshape(()))


# speedup vs baseline: 1.6599x; 1.6599x over previous
import functools

import jax
import jax.numpy as jnp
from jax import lax
from jax.experimental import pallas as pl
from jax.experimental.pallas import tpu as pltpu
from jax.experimental.pallas import tpu_sc as plsc

N, D, HW = 8, 32, 1024
C_IMG = 3
K = 8192
KT = 512
NKT = K // KT
P = N * HW
COMMITMENT_WEIGHT = 0.25
VQLOSS_WEIGHT = 1.0

SC_NC, SC_NS = 2, 16
SC_NW = SC_NC * SC_NS
ROWS_PER_W = P // SC_NW


def _argmin_body(lat_ref, cb_ref, idx_ref, bv_ref, bi_ref):
    k = pl.program_id(1)
    latn = lat_ref[0]
    cb = cb_ref[...]
    c2 = jnp.sum(cb * cb, axis=1, keepdims=True)
    xc = lax.dot_general(cb, latn, (((1,), (0,)), ((), ())),
                         preferred_element_type=jnp.float32,
                         precision=lax.Precision.DEFAULT)
    s = xc - 0.5 * c2
    m = jnp.max(s, axis=0, keepdims=True)
    eq = (s == m).astype(jnp.float32)
    iota = lax.broadcasted_iota(jnp.int32, (1, KT), 1).astype(jnp.float32)
    loc = lax.dot_general(iota, eq, (((1,), (0,)), ((), ())),
                          preferred_element_type=jnp.float32,
                          precision=lax.Precision.HIGHEST)

    @pl.when(k == 0)
    def _init():
        bv_ref[...] = jnp.full((1, HW), -jnp.inf, jnp.float32)
        bi_ref[...] = jnp.zeros((1, HW), jnp.float32)

    upd = m > bv_ref[...]
    bi_ref[...] = jnp.where(upd, loc + jnp.float32(k * KT), bi_ref[...])
    bv_ref[...] = jnp.where(upd, m, bv_ref[...])

    @pl.when(k == NKT - 1)
    def _emit():
        idx_ref[0] = jnp.clip(bi_ref[...], 0.0, float(K - 1)).astype(jnp.int32)


def _nearest_indices(lat3, codebook, interpret=False):
    return pl.pallas_call(
        _argmin_body,
        grid=(N, NKT),
        in_specs=[
            pl.BlockSpec((1, D, HW), lambda n, k: (n, 0, 0)),
            pl.BlockSpec((KT, D), lambda n, k: (k, 0)),
        ],
        out_specs=pl.BlockSpec((1, 1, HW), lambda n, k: (n, 0, 0)),
        out_shape=jax.ShapeDtypeStruct((N, 1, HW), jnp.int32),
        scratch_shapes=[
            pltpu.VMEM((1, HW), jnp.float32),
            pltpu.VMEM((1, HW), jnp.float32),
        ],
        interpret=interpret,
    )(lat3, codebook)


def _sc_gather(idx_flat, codebook):
    mesh = plsc.VectorSubcoreMesh(core_axis_name="c", subcore_axis_name="s")

    @functools.partial(
        pl.kernel,
        mesh=mesh,
        compiler_params=pltpu.CompilerParams(use_tc_tiling_on_sc=False),
        out_type=jax.ShapeDtypeStruct((P, D), jnp.float32),
        scratch_types=[
            pltpu.VMEM((ROWS_PER_W,), jnp.int32),
            pltpu.VMEM((ROWS_PER_W, D), jnp.float32),
            pltpu.SemaphoreType.DMA,
        ],
    )
    def gather_kernel(idx_hbm, cb_hbm, out_hbm, idx_v, rows_v, sem):
        wid = lax.axis_index("s") * SC_NC + lax.axis_index("c")
        base = wid * ROWS_PER_W
        pltpu.sync_copy(idx_hbm.at[pl.ds(base, ROWS_PER_W)], idx_v)
        pltpu.async_copy(cb_hbm.at[idx_v], rows_v, sem).wait()
        pltpu.sync_copy(rows_v, out_hbm.at[pl.ds(base, ROWS_PER_W)])

    return gather_kernel(idx_flat, codebook)


def _loss_body(lat_ref, q_ref, tgt_ref, w_ref, b_ref,
               loss_ref, vq_ref, rec_ref):
    eye = (lax.broadcasted_iota(jnp.int32, (D, D), 0)
           == lax.broadcasted_iota(jnp.int32, (D, D), 1))
    w = w_ref[...]
    b = b_ref[...]
    cb_sum = jnp.float32(0.0)
    rec_sum = jnp.float32(0.0)
    for n in range(N):
        latn = lat_ref[n]
        qn = q_ref[pl.ds(n * HW, HW), :]
        m = lax.dot_general(latn, qn, (((1,), (0,)), ((), ())),
                            preferred_element_type=jnp.float32,
                            precision=lax.Precision.HIGHEST)
        cross = jnp.sum(jnp.where(eye, m, 0.0))
        cb_sum = cb_sum + jnp.sum(latn * latn) + jnp.sum(qn * qn) - 2.0 * cross
        out = lax.dot_general(w, qn, (((1,), (1,)), ((), ())),
                              preferred_element_type=jnp.float32,
                              precision=lax.Precision.HIGHEST) + b
        r = out - tgt_ref[n]
        rec_sum = rec_sum + jnp.sum(r * r)
    cb_mean = cb_sum / jnp.float32(P * D)
    vq = (VQLOSS_WEIGHT + VQLOSS_WEIGHT * COMMITMENT_WEIGHT) * cb_mean
    rec = rec_sum / jnp.float32(P * C_IMG)
    loss_ref[0, 0] = vq + rec
    vq_ref[0, 0] = vq
    rec_ref[0, 0] = rec


def _losses(lat3, q, tgt3, w_dec, b_dec2, interpret=False):
    smem_out = pl.BlockSpec(memory_space=pltpu.SMEM)
    return pl.pallas_call(
        _loss_body,
        in_specs=[pl.BlockSpec(memory_space=pltpu.VMEM)] * 5,
        out_specs=(smem_out, smem_out, smem_out),
        out_shape=(
            jax.ShapeDtypeStruct((1, 1), jnp.float32),
            jax.ShapeDtypeStruct((1, 1), jnp.float32),
            jax.ShapeDtypeStruct((1, 1), jnp.float32),
        ),
        interpret=interpret,
    )(lat3, q, tgt3, w_dec, b_dec2)


def kernel(latents, target, codebook, W_dec, b_dec):
    lat3 = latents.reshape(N, D, HW)
    tgt3 = target.reshape(N, C_IMG, HW)
    idx = _nearest_indices(lat3, codebook)
    q = _sc_gather(idx.reshape(P), codebook)
    loss, vq, rec = _losses(lat3, q, tgt3, W_dec, b_dec.reshape(C_IMG, 1))
    return (loss.reshape(()), vq.reshape(()), rec.reshape(()))


# speedup vs baseline: 2.7203x; 1.6388x over previous
import functools

import jax
import jax.numpy as jnp
from jax import lax
from jax.experimental import pallas as pl
from jax.experimental.pallas import tpu as pltpu
from jax.experimental.pallas import tpu_sc as plsc

N, D, HW = 8, 32, 1024
C_IMG = 3
K = 8192
KT = 512
NKT = K // KT
P = N * HW
COMMITMENT_WEIGHT = 0.25
VQLOSS_WEIGHT = 1.0

SC_NC, SC_NS = 2, 16
SC_NW = SC_NC * SC_NS
ROWS_PER_W = P // SC_NW


def _argmin_body(lat_ref, cb_ref, idx_ref, bv_ref, bi_ref):
    k = pl.program_id(1)
    latn = lat_ref[0]
    cb = cb_ref[...]
    c2 = jnp.sum(cb * cb, axis=1, keepdims=True)
    xc = lax.dot_general(cb, latn, (((1,), (0,)), ((), ())),
                         preferred_element_type=jnp.float32,
                         precision=lax.Precision.DEFAULT)
    s = xc - 0.5 * c2
    m = jnp.max(s, axis=0, keepdims=True)
    riota = lax.broadcasted_iota(jnp.int32, (KT, HW), 0)
    loc = jnp.max(jnp.where(s == m, riota, 0), axis=0, keepdims=True)

    @pl.when(k == 0)
    def _init():
        bv_ref[...] = jnp.full((1, HW), -jnp.inf, jnp.float32)
        bi_ref[...] = jnp.zeros((1, HW), jnp.int32)

    upd = m > bv_ref[...]
    bi_ref[...] = jnp.where(upd, loc + k * KT, bi_ref[...])
    bv_ref[...] = jnp.where(upd, m, bv_ref[...])

    @pl.when(k == NKT - 1)
    def _emit():
        idx_ref[0] = bi_ref[...]


def _nearest_indices(lat3, codebook, interpret=False):
    return pl.pallas_call(
        _argmin_body,
        grid=(N, NKT),
        in_specs=[
            pl.BlockSpec((1, D, HW), lambda n, k: (n, 0, 0)),
            pl.BlockSpec((KT, D), lambda n, k: (k, 0)),
        ],
        out_specs=pl.BlockSpec((1, 1, HW), lambda n, k: (n, 0, 0)),
        out_shape=jax.ShapeDtypeStruct((N, 1, HW), jnp.int32),
        scratch_shapes=[
            pltpu.VMEM((1, HW), jnp.float32),
            pltpu.VMEM((1, HW), jnp.int32),
        ],
        interpret=interpret,
    )(lat3, codebook)


def _sc_gather(idx_flat, codebook):
    mesh = plsc.VectorSubcoreMesh(core_axis_name="c", subcore_axis_name="s")

    @functools.partial(
        pl.kernel,
        mesh=mesh,
        compiler_params=pltpu.CompilerParams(use_tc_tiling_on_sc=False),
        out_type=jax.ShapeDtypeStruct((P, D), jnp.float32),
        scratch_types=[
            pltpu.VMEM((ROWS_PER_W,), jnp.int32),
            pltpu.VMEM((ROWS_PER_W, D), jnp.float32),
            pltpu.SemaphoreType.DMA,
        ],
    )
    def gather_kernel(idx_hbm, cb_hbm, out_hbm, idx_v, rows_v, sem):
        wid = lax.axis_index("s") * SC_NC + lax.axis_index("c")
        base = wid * ROWS_PER_W
        pltpu.sync_copy(idx_hbm.at[pl.ds(base, ROWS_PER_W)], idx_v)
        pltpu.async_copy(cb_hbm.at[idx_v], rows_v, sem).wait()
        pltpu.sync_copy(rows_v, out_hbm.at[pl.ds(base, ROWS_PER_W)])

    return gather_kernel(idx_flat, codebook)


def _loss_body(lat_ref, q_ref, tgt_ref, w_ref, b_ref,
               loss_ref, vq_ref, rec_ref):
    eye = (lax.broadcasted_iota(jnp.int32, (D, D), 0)
           == lax.broadcasted_iota(jnp.int32, (D, D), 1))
    w = w_ref[...]
    b = b_ref[...]
    cb_sum = jnp.float32(0.0)
    rec_sum = jnp.float32(0.0)
    for n in range(N):
        latn = lat_ref[n]
        qn = q_ref[pl.ds(n * HW, HW), :]
        m = lax.dot_general(latn, qn, (((1,), (0,)), ((), ())),
                            preferred_element_type=jnp.float32,
                            precision=lax.Precision.HIGHEST)
        cross = jnp.sum(jnp.where(eye, m, 0.0))
        cb_sum = cb_sum + jnp.sum(latn * latn) + jnp.sum(qn * qn) - 2.0 * cross
        out = lax.dot_general(w, qn, (((1,), (1,)), ((), ())),
                              preferred_element_type=jnp.float32,
                              precision=lax.Precision.HIGHEST) + b
        r = out - tgt_ref[n]
        rec_sum = rec_sum + jnp.sum(r * r)
    cb_mean = cb_sum / jnp.float32(P * D)
    vq = (VQLOSS_WEIGHT + VQLOSS_WEIGHT * COMMITMENT_WEIGHT) * cb_mean
    rec = rec_sum / jnp.float32(P * C_IMG)
    loss_ref[0, 0] = vq + rec
    vq_ref[0, 0] = vq
    rec_ref[0, 0] = rec


def _losses(lat3, q, tgt3, w_dec, b_dec2, interpret=False):
    smem_out = pl.BlockSpec(memory_space=pltpu.SMEM)
    return pl.pallas_call(
        _loss_body,
        in_specs=[pl.BlockSpec(memory_space=pltpu.VMEM)] * 5,
        out_specs=(smem_out, smem_out, smem_out),
        out_shape=(
            jax.ShapeDtypeStruct((1, 1), jnp.float32),
            jax.ShapeDtypeStruct((1, 1), jnp.float32),
            jax.ShapeDtypeStruct((1, 1), jnp.float32),
        ),
        interpret=interpret,
    )(lat3, q, tgt3, w_dec, b_dec2)


def kernel(latents, target, codebook, W_dec, b_dec):
    lat3 = latents.reshape(N, D, HW)
    tgt3 = target.reshape(N, C_IMG, HW)
    idx = _nearest_indices(lat3, codebook)
    q = _sc_gather(idx.reshape(P), codebook)
    loss, vq, rec = _losses(lat3, q, tgt3, W_dec, b_dec.reshape(C_IMG, 1))
    return (loss.reshape(()), vq.reshape(()), rec.reshape(()))


# speedup vs baseline: 3.2044x; 1.1780x over previous
import functools

import jax
import jax.numpy as jnp
from jax import lax
from jax.experimental import pallas as pl
from jax.experimental.pallas import tpu as pltpu
from jax.experimental.pallas import tpu_sc as plsc

N, D, HW = 8, 32, 1024
C_IMG = 3
K = 8192
KT = 1024
NKT = K // KT
P = N * HW
COMMITMENT_WEIGHT = 0.25
VQLOSS_WEIGHT = 1.0

SC_NC, SC_NS = 2, 16
SC_NW = SC_NC * SC_NS
ROWS_PER_W = P // SC_NW


def _argmin_body(lat_ref, cb_ref, idx_ref, bv_ref, bi_ref):
    k = pl.program_id(1)
    latn = lat_ref[0]
    cb = cb_ref[...]
    c2 = jnp.sum(cb * cb, axis=1, keepdims=True)
    xc = lax.dot_general(cb, latn, (((1,), (0,)), ((), ())),
                         preferred_element_type=jnp.float32,
                         precision=lax.Precision.DEFAULT)
    s = xc - 0.5 * c2
    m = jnp.max(s, axis=0, keepdims=True)
    riota = lax.broadcasted_iota(jnp.int32, (KT, HW), 0)
    loc = jnp.max(jnp.where(s == m, riota, 0), axis=0, keepdims=True)

    @pl.when(k == 0)
    def _init():
        bv_ref[...] = jnp.full((1, HW), -jnp.inf, jnp.float32)
        bi_ref[...] = jnp.zeros((1, HW), jnp.int32)

    upd = m > bv_ref[...]
    bi_ref[...] = jnp.where(upd, loc + k * KT, bi_ref[...])
    bv_ref[...] = jnp.where(upd, m, bv_ref[...])

    @pl.when(k == NKT - 1)
    def _emit():
        idx_ref[0] = bi_ref[...]


def _nearest_indices(lat3, codebook, interpret=False):
    return pl.pallas_call(
        _argmin_body,
        grid=(N, NKT),
        in_specs=[
            pl.BlockSpec((1, D, HW), lambda n, k: (n, 0, 0)),
            pl.BlockSpec((KT, D), lambda n, k: (k, 0)),
        ],
        out_specs=pl.BlockSpec((1, 1, HW), lambda n, k: (n, 0, 0)),
        out_shape=jax.ShapeDtypeStruct((N, 1, HW), jnp.int32),
        scratch_shapes=[
            pltpu.VMEM((1, HW), jnp.float32),
            pltpu.VMEM((1, HW), jnp.int32),
        ],
        interpret=interpret,
    )(lat3, codebook)


def _sc_gather(idx_flat, codebook):
    mesh = plsc.VectorSubcoreMesh(core_axis_name="c", subcore_axis_name="s")

    @functools.partial(
        pl.kernel,
        mesh=mesh,
        compiler_params=pltpu.CompilerParams(use_tc_tiling_on_sc=False),
        out_type=jax.ShapeDtypeStruct((P, D), jnp.float32),
        scratch_types=[
            pltpu.VMEM((ROWS_PER_W,), jnp.int32),
            pltpu.VMEM((ROWS_PER_W, D), jnp.float32),
            pltpu.SemaphoreType.DMA,
        ],
    )
    def gather_kernel(idx_hbm, cb_hbm, out_hbm, idx_v, rows_v, sem):
        wid = lax.axis_index("s") * SC_NC + lax.axis_index("c")
        base = wid * ROWS_PER_W
        pltpu.sync_copy(idx_hbm.at[pl.ds(base, ROWS_PER_W)], idx_v)
        pltpu.async_copy(cb_hbm.at[idx_v], rows_v, sem).wait()
        pltpu.sync_copy(rows_v, out_hbm.at[pl.ds(base, ROWS_PER_W)])

    return gather_kernel(idx_flat, codebook)


def _loss_body(lat_ref, q_ref, tgt_ref, w_ref, b_ref,
               loss_ref, vq_ref, rec_ref):
    eye = (lax.broadcasted_iota(jnp.int32, (D, D), 0)
           == lax.broadcasted_iota(jnp.int32, (D, D), 1))
    w = w_ref[...]
    b = b_ref[...]
    cb_sum = jnp.float32(0.0)
    rec_sum = jnp.float32(0.0)
    for n in range(N):
        latn = lat_ref[n]
        qn = q_ref[pl.ds(n * HW, HW), :]
        m = lax.dot_general(latn, qn, (((1,), (0,)), ((), ())),
                            preferred_element_type=jnp.float32,
                            precision=lax.Precision.HIGHEST)
        cross = jnp.sum(jnp.where(eye, m, 0.0))
        cb_sum = cb_sum + jnp.sum(latn * latn) + jnp.sum(qn * qn) - 2.0 * cross
        out = lax.dot_general(w, qn, (((1,), (1,)), ((), ())),
                              preferred_element_type=jnp.float32,
                              precision=lax.Precision.HIGHEST) + b
        r = out - tgt_ref[n]
        rec_sum = rec_sum + jnp.sum(r * r)
    cb_mean = cb_sum / jnp.float32(P * D)
    vq = (VQLOSS_WEIGHT + VQLOSS_WEIGHT * COMMITMENT_WEIGHT) * cb_mean
    rec = rec_sum / jnp.float32(P * C_IMG)
    loss_ref[0, 0] = vq + rec
    vq_ref[0, 0] = vq
    rec_ref[0, 0] = rec


def _losses(lat3, q, tgt3, w_dec, b_dec2, interpret=False):
    smem_out = pl.BlockSpec(memory_space=pltpu.SMEM)
    return pl.pallas_call(
        _loss_body,
        in_specs=[pl.BlockSpec(memory_space=pltpu.VMEM)] * 5,
        out_specs=(smem_out, smem_out, smem_out),
        out_shape=(
            jax.ShapeDtypeStruct((1, 1), jnp.float32),
            jax.ShapeDtypeStruct((1, 1), jnp.float32),
            jax.ShapeDtypeStruct((1, 1), jnp.float32),
        ),
        interpret=interpret,
    )(lat3, q, tgt3, w_dec, b_dec2)


def kernel(latents, target, codebook, W_dec, b_dec):
    lat3 = latents.reshape(N, D, HW)
    tgt3 = target.reshape(N, C_IMG, HW)
    idx = _nearest_indices(lat3, codebook)
    q = _sc_gather(idx.reshape(P), codebook)
    loss, vq, rec = _losses(lat3, q, tgt3, W_dec, b_dec.reshape(C_IMG, 1))
    return (loss.reshape(()), vq.reshape(()), rec.reshape(()))


# speedup vs baseline: 3.2550x; 1.0158x over previous
import functools

import jax
import jax.numpy as jnp
from jax import lax
from jax.experimental import pallas as pl
from jax.experimental.pallas import tpu as pltpu
from jax.experimental.pallas import tpu_sc as plsc

N, D, HW = 8, 32, 1024
C_IMG = 3
K = 8192
KT = 2048
NKT = K // KT
P = N * HW
COMMITMENT_WEIGHT = 0.25
VQLOSS_WEIGHT = 1.0

SC_NC, SC_NS = 2, 16
SC_NW = SC_NC * SC_NS
ROWS_PER_W = P // SC_NW


def _argmin_body(lat_ref, cb_ref, idx_ref, bv_ref, bi_ref):
    k = pl.program_id(1)
    latn = lat_ref[0]
    cb = cb_ref[...]
    c2 = jnp.sum(cb * cb, axis=1, keepdims=True)
    xc = lax.dot_general(cb, latn, (((1,), (0,)), ((), ())),
                         preferred_element_type=jnp.float32,
                         precision=lax.Precision.DEFAULT)
    s = xc - 0.5 * c2
    m = jnp.max(s, axis=0, keepdims=True)
    riota = lax.broadcasted_iota(jnp.int32, (KT, HW), 0)
    loc = jnp.max(jnp.where(s == m, riota, 0), axis=0, keepdims=True)

    @pl.when(k == 0)
    def _init():
        bv_ref[...] = jnp.full((1, HW), -jnp.inf, jnp.float32)
        bi_ref[...] = jnp.zeros((1, HW), jnp.int32)

    upd = m > bv_ref[...]
    bi_ref[...] = jnp.where(upd, loc + k * KT, bi_ref[...])
    bv_ref[...] = jnp.where(upd, m, bv_ref[...])

    @pl.when(k == NKT - 1)
    def _emit():
        idx_ref[0] = bi_ref[...]


def _nearest_indices(lat3, codebook, interpret=False):
    return pl.pallas_call(
        _argmin_body,
        grid=(N, NKT),
        in_specs=[
            pl.BlockSpec((1, D, HW), lambda n, k: (n, 0, 0)),
            pl.BlockSpec((KT, D), lambda n, k: (k, 0)),
        ],
        out_specs=pl.BlockSpec((1, 1, HW), lambda n, k: (n, 0, 0)),
        out_shape=jax.ShapeDtypeStruct((N, 1, HW), jnp.int32),
        scratch_shapes=[
            pltpu.VMEM((1, HW), jnp.float32),
            pltpu.VMEM((1, HW), jnp.int32),
        ],
        interpret=interpret,
    )(lat3, codebook)


def _sc_gather(idx_flat, codebook):
    mesh = plsc.VectorSubcoreMesh(core_axis_name="c", subcore_axis_name="s")

    @functools.partial(
        pl.kernel,
        mesh=mesh,
        compiler_params=pltpu.CompilerParams(use_tc_tiling_on_sc=False),
        out_type=jax.ShapeDtypeStruct((P, D), jnp.float32),
        scratch_types=[
            pltpu.VMEM((ROWS_PER_W,), jnp.int32),
            pltpu.VMEM((ROWS_PER_W, D), jnp.float32),
            pltpu.SemaphoreType.DMA,
        ],
    )
    def gather_kernel(idx_hbm, cb_hbm, out_hbm, idx_v, rows_v, sem):
        wid = lax.axis_index("s") * SC_NC + lax.axis_index("c")
        base = wid * ROWS_PER_W
        pltpu.sync_copy(idx_hbm.at[pl.ds(base, ROWS_PER_W)], idx_v)
        pltpu.async_copy(cb_hbm.at[idx_v], rows_v, sem).wait()
        pltpu.sync_copy(rows_v, out_hbm.at[pl.ds(base, ROWS_PER_W)])

    return gather_kernel(idx_flat, codebook)


def _loss_body(lat_ref, q_ref, tgt_ref, w_ref, b_ref,
               loss_ref, vq_ref, rec_ref):
    eye = (lax.broadcasted_iota(jnp.int32, (D, D), 0)
           == lax.broadcasted_iota(jnp.int32, (D, D), 1))
    w = w_ref[...]
    b = b_ref[...]
    cb_sum = jnp.float32(0.0)
    rec_sum = jnp.float32(0.0)
    for n in range(N):
        latn = lat_ref[n]
        qn = q_ref[pl.ds(n * HW, HW), :]
        m = lax.dot_general(latn, qn, (((1,), (0,)), ((), ())),
                            preferred_element_type=jnp.float32,
                            precision=lax.Precision.HIGHEST)
        cross = jnp.sum(jnp.where(eye, m, 0.0))
        cb_sum = cb_sum + jnp.sum(latn * latn) + jnp.sum(qn * qn) - 2.0 * cross
        out = lax.dot_general(w, qn, (((1,), (1,)), ((), ())),
                              preferred_element_type=jnp.float32,
                              precision=lax.Precision.HIGHEST) + b
        r = out - tgt_ref[n]
        rec_sum = rec_sum + jnp.sum(r * r)
    cb_mean = cb_sum / jnp.float32(P * D)
    vq = (VQLOSS_WEIGHT + VQLOSS_WEIGHT * COMMITMENT_WEIGHT) * cb_mean
    rec = rec_sum / jnp.float32(P * C_IMG)
    loss_ref[0, 0] = vq + rec
    vq_ref[0, 0] = vq
    rec_ref[0, 0] = rec


def _losses(lat3, q, tgt3, w_dec, b_dec2, interpret=False):
    smem_out = pl.BlockSpec(memory_space=pltpu.SMEM)
    return pl.pallas_call(
        _loss_body,
        in_specs=[pl.BlockSpec(memory_space=pltpu.VMEM)] * 5,
        out_specs=(smem_out, smem_out, smem_out),
        out_shape=(
            jax.ShapeDtypeStruct((1, 1), jnp.float32),
            jax.ShapeDtypeStruct((1, 1), jnp.float32),
            jax.ShapeDtypeStruct((1, 1), jnp.float32),
        ),
        interpret=interpret,
    )(lat3, q, tgt3, w_dec, b_dec2)


def kernel(latents, target, codebook, W_dec, b_dec):
    lat3 = latents.reshape(N, D, HW)
    tgt3 = target.reshape(N, C_IMG, HW)
    idx = _nearest_indices(lat3, codebook)
    q = _sc_gather(idx.reshape(P), codebook)
    loss, vq, rec = _losses(lat3, q, tgt3, W_dec, b_dec.reshape(C_IMG, 1))
    return (loss.reshape(()), vq.reshape(()), rec.reshape(()))


# speedup vs baseline: 4.2369x; 1.3016x over previous
import functools

import jax
import jax.numpy as jnp
from jax import lax
from jax.experimental import pallas as pl
from jax.experimental.pallas import tpu as pltpu
from jax.experimental.pallas import tpu_sc as plsc

N, D, HW = 8, 32, 1024
C_IMG = 3
K = 8192
KT = 2048
NKT = K // KT
P = N * HW
COMMITMENT_WEIGHT = 0.25
VQLOSS_WEIGHT = 1.0

SC_NC, SC_NS = 2, 16
SC_NW = SC_NC * SC_NS
ROWS_PER_W = P // SC_NW


def _argmin_body(lat_ref, cb_ref, idx_ref, bv_ref, bi_ref):
    k = pl.program_id(1)
    latn = lat_ref[0]
    cb = cb_ref[...]
    c2 = jnp.sum(cb * cb, axis=1, keepdims=True)
    xc = lax.dot_general(cb, latn, (((1,), (0,)), ((), ())),
                         preferred_element_type=jnp.float32,
                         precision=lax.Precision.DEFAULT)
    s = xc - 0.5 * c2
    m = jnp.max(s, axis=0, keepdims=True)
    loc = jnp.argmax(s, axis=0).astype(jnp.int32)[None, :]

    @pl.when(k == 0)
    def _init():
        bv_ref[...] = jnp.full((1, HW), -jnp.inf, jnp.float32)
        bi_ref[...] = jnp.zeros((1, HW), jnp.int32)

    upd = m > bv_ref[...]
    bi_ref[...] = jnp.where(upd, loc + k * KT, bi_ref[...])
    bv_ref[...] = jnp.where(upd, m, bv_ref[...])

    @pl.when(k == NKT - 1)
    def _emit():
        idx_ref[0] = bi_ref[...]


def _nearest_indices(lat3, codebook, interpret=False):
    return pl.pallas_call(
        _argmin_body,
        grid=(N, NKT),
        in_specs=[
            pl.BlockSpec((1, D, HW), lambda n, k: (n, 0, 0)),
            pl.BlockSpec((KT, D), lambda n, k: (k, 0)),
        ],
        out_specs=pl.BlockSpec((1, 1, HW), lambda n, k: (n, 0, 0)),
        out_shape=jax.ShapeDtypeStruct((N, 1, HW), jnp.int32),
        scratch_shapes=[
            pltpu.VMEM((1, HW), jnp.float32),
            pltpu.VMEM((1, HW), jnp.int32),
        ],
        interpret=interpret,
    )(lat3, codebook)


def _sc_gather(idx_flat, codebook):
    mesh = plsc.VectorSubcoreMesh(core_axis_name="c", subcore_axis_name="s")

    @functools.partial(
        pl.kernel,
        mesh=mesh,
        compiler_params=pltpu.CompilerParams(use_tc_tiling_on_sc=False),
        out_type=jax.ShapeDtypeStruct((P, D), jnp.float32),
        scratch_types=[
            pltpu.VMEM((ROWS_PER_W,), jnp.int32),
            pltpu.VMEM((ROWS_PER_W, D), jnp.float32),
            pltpu.SemaphoreType.DMA,
        ],
    )
    def gather_kernel(idx_hbm, cb_hbm, out_hbm, idx_v, rows_v, sem):
        wid = lax.axis_index("s") * SC_NC + lax.axis_index("c")
        base = wid * ROWS_PER_W
        pltpu.sync_copy(idx_hbm.at[pl.ds(base, ROWS_PER_W)], idx_v)
        pltpu.async_copy(cb_hbm.at[idx_v], rows_v, sem).wait()
        pltpu.sync_copy(rows_v, out_hbm.at[pl.ds(base, ROWS_PER_W)])

    return gather_kernel(idx_flat, codebook)


def _loss_body(lat_ref, q_ref, tgt_ref, w_ref, b_ref,
               loss_ref, vq_ref, rec_ref):
    eye = (lax.broadcasted_iota(jnp.int32, (D, D), 0)
           == lax.broadcasted_iota(jnp.int32, (D, D), 1))
    w = w_ref[...]
    b = b_ref[...]
    cb_sum = jnp.float32(0.0)
    rec_sum = jnp.float32(0.0)
    for n in range(N):
        latn = lat_ref[n]
        qn = q_ref[pl.ds(n * HW, HW), :]
        m = lax.dot_general(latn, qn, (((1,), (0,)), ((), ())),
                            preferred_element_type=jnp.float32,
                            precision=lax.Precision.HIGHEST)
        cross = jnp.sum(jnp.where(eye, m, 0.0))
        cb_sum = cb_sum + jnp.sum(latn * latn) + jnp.sum(qn * qn) - 2.0 * cross
        out = lax.dot_general(w, qn, (((1,), (1,)), ((), ())),
                              preferred_element_type=jnp.float32,
                              precision=lax.Precision.HIGHEST) + b
        r = out - tgt_ref[n]
        rec_sum = rec_sum + jnp.sum(r * r)
    cb_mean = cb_sum / jnp.float32(P * D)
    vq = (VQLOSS_WEIGHT + VQLOSS_WEIGHT * COMMITMENT_WEIGHT) * cb_mean
    rec = rec_sum / jnp.float32(P * C_IMG)
    loss_ref[0, 0] = vq + rec
    vq_ref[0, 0] = vq
    rec_ref[0, 0] = rec


def _losses(lat3, q, tgt3, w_dec, b_dec2, interpret=False):
    smem_out = pl.BlockSpec(memory_space=pltpu.SMEM)
    return pl.pallas_call(
        _loss_body,
        in_specs=[pl.BlockSpec(memory_space=pltpu.VMEM)] * 5,
        out_specs=(smem_out, smem_out, smem_out),
        out_shape=(
            jax.ShapeDtypeStruct((1, 1), jnp.float32),
            jax.ShapeDtypeStruct((1, 1), jnp.float32),
            jax.ShapeDtypeStruct((1, 1), jnp.float32),
        ),
        interpret=interpret,
    )(lat3, q, tgt3, w_dec, b_dec2)


def kernel(latents, target, codebook, W_dec, b_dec):
    lat3 = latents.reshape(N, D, HW)
    tgt3 = target.reshape(N, C_IMG, HW)
    idx = _nearest_indices(lat3, codebook)
    q = _sc_gather(idx.reshape(P), codebook)
    loss, vq, rec = _losses(lat3, q, tgt3, W_dec, b_dec.reshape(C_IMG, 1))
    return (loss.reshape(()), vq.reshape(()), rec.reshape(()))


# speedup vs baseline: 4.6347x; 1.0939x over previous
import functools

import jax
import jax.numpy as jnp
from jax import lax
from jax.experimental import pallas as pl
from jax.experimental.pallas import tpu as pltpu
from jax.experimental.pallas import tpu_sc as plsc

N, D, HW = 8, 32, 1024
C_IMG = 3
K = 8192
PT = 512
P = N * HW
COMMITMENT_WEIGHT = 0.25
VQLOSS_WEIGHT = 1.0

SC_NC, SC_NS = 2, 16
SC_NW = SC_NC * SC_NS
ROWS_PER_W = P // SC_NW


def _argmin_body(lat_ref, cb_ref, idx_ref):
    latp = lat_ref[0]
    cb = cb_ref[...]
    c2 = jnp.sum(cb * cb, axis=1, keepdims=True)
    xc = lax.dot_general(cb, latp, (((1,), (0,)), ((), ())),
                         preferred_element_type=jnp.float32,
                         precision=lax.Precision.DEFAULT)
    s = xc - 0.5 * c2
    idx_ref[0] = jnp.argmax(s, axis=0).astype(jnp.int32)[None, :]


def _nearest_indices(lat3, codebook, interpret=False):
    return pl.pallas_call(
        _argmin_body,
        grid=(N, HW // PT),
        in_specs=[
            pl.BlockSpec((1, D, PT), lambda n, j: (n, 0, j)),
            pl.BlockSpec((K, D), lambda n, j: (0, 0)),
        ],
        out_specs=pl.BlockSpec((1, 1, PT), lambda n, j: (n, 0, j)),
        out_shape=jax.ShapeDtypeStruct((N, 1, HW), jnp.int32),
        interpret=interpret,
    )(lat3, codebook)


def _sc_gather(idx_flat, codebook):
    mesh = plsc.VectorSubcoreMesh(core_axis_name="c", subcore_axis_name="s")

    @functools.partial(
        pl.kernel,
        mesh=mesh,
        compiler_params=pltpu.CompilerParams(use_tc_tiling_on_sc=False),
        out_type=jax.ShapeDtypeStruct((P, D), jnp.float32),
        scratch_types=[
            pltpu.VMEM((ROWS_PER_W,), jnp.int32),
            pltpu.VMEM((ROWS_PER_W, D), jnp.float32),
            pltpu.SemaphoreType.DMA,
        ],
    )
    def gather_kernel(idx_hbm, cb_hbm, out_hbm, idx_v, rows_v, sem):
        wid = lax.axis_index("s") * SC_NC + lax.axis_index("c")
        base = wid * ROWS_PER_W
        pltpu.sync_copy(idx_hbm.at[pl.ds(base, ROWS_PER_W)], idx_v)
        pltpu.async_copy(cb_hbm.at[idx_v], rows_v, sem).wait()
        pltpu.sync_copy(rows_v, out_hbm.at[pl.ds(base, ROWS_PER_W)])

    return gather_kernel(idx_flat, codebook)


def _loss_body(lat_ref, q_ref, tgt_ref, w_ref, b_ref,
               loss_ref, vq_ref, rec_ref):
    eye = (lax.broadcasted_iota(jnp.int32, (D, D), 0)
           == lax.broadcasted_iota(jnp.int32, (D, D), 1))
    w = w_ref[...]
    b = b_ref[...]
    cb_sum = jnp.float32(0.0)
    rec_sum = jnp.float32(0.0)
    for n in range(N):
        latn = lat_ref[n]
        qn = q_ref[pl.ds(n * HW, HW), :]
        m = lax.dot_general(latn, qn, (((1,), (0,)), ((), ())),
                            preferred_element_type=jnp.float32,
                            precision=lax.Precision.HIGHEST)
        cross = jnp.sum(jnp.where(eye, m, 0.0))
        cb_sum = cb_sum + jnp.sum(latn * latn) + jnp.sum(qn * qn) - 2.0 * cross
        out = lax.dot_general(w, qn, (((1,), (1,)), ((), ())),
                              preferred_element_type=jnp.float32,
                              precision=lax.Precision.HIGHEST) + b
        r = out - tgt_ref[n]
        rec_sum = rec_sum + jnp.sum(r * r)
    cb_mean = cb_sum / jnp.float32(P * D)
    vq = (VQLOSS_WEIGHT + VQLOSS_WEIGHT * COMMITMENT_WEIGHT) * cb_mean
    rec = rec_sum / jnp.float32(P * C_IMG)
    loss_ref[0, 0] = vq + rec
    vq_ref[0, 0] = vq
    rec_ref[0, 0] = rec


def _losses(lat3, q, tgt3, w_dec, b_dec2, interpret=False):
    smem_out = pl.BlockSpec(memory_space=pltpu.SMEM)
    return pl.pallas_call(
        _loss_body,
        in_specs=[pl.BlockSpec(memory_space=pltpu.VMEM)] * 5,
        out_specs=(smem_out, smem_out, smem_out),
        out_shape=(
            jax.ShapeDtypeStruct((1, 1), jnp.float32),
            jax.ShapeDtypeStruct((1, 1), jnp.float32),
            jax.ShapeDtypeStruct((1, 1), jnp.float32),
        ),
        interpret=interpret,
    )(lat3, q, tgt3, w_dec, b_dec2)


def kernel(latents, target, codebook, W_dec, b_dec):
    lat3 = latents.reshape(N, D, HW)
    tgt3 = target.reshape(N, C_IMG, HW)
    idx = _nearest_indices(lat3, codebook)
    q = _sc_gather(idx.reshape(P), codebook)
    loss, vq, rec = _losses(lat3, q, tgt3, W_dec, b_dec.reshape(C_IMG, 1))
    return (loss.reshape(()), vq.reshape(()), rec.reshape(()))


# speedup vs baseline: 4.8428x; 1.0449x over previous
import functools

import jax
import jax.numpy as jnp
from jax import lax
from jax.experimental import pallas as pl
from jax.experimental.pallas import tpu as pltpu
from jax.experimental.pallas import tpu_sc as plsc

N, D, HW = 8, 32, 1024
C_IMG = 3
K = 8192
PT = 1024
P = N * HW
COMMITMENT_WEIGHT = 0.25
VQLOSS_WEIGHT = 1.0

SC_NC, SC_NS = 2, 16
SC_NW = SC_NC * SC_NS
ROWS_PER_W = P // SC_NW


def _argmin_body(lat_ref, cb_ref, idx_ref):
    latp = lat_ref[0]
    cb = cb_ref[...]
    c2 = jnp.sum(cb * cb, axis=1, keepdims=True)
    xc = lax.dot_general(cb, latp, (((1,), (0,)), ((), ())),
                         preferred_element_type=jnp.float32,
                         precision=lax.Precision.DEFAULT)
    s = xc - 0.5 * c2
    idx_ref[0] = jnp.argmax(s, axis=0).astype(jnp.int32)[None, :]


def _nearest_indices(lat3, codebook, interpret=False):
    return pl.pallas_call(
        _argmin_body,
        grid=(N, HW // PT),
        in_specs=[
            pl.BlockSpec((1, D, PT), lambda n, j: (n, 0, j)),
            pl.BlockSpec((K, D), lambda n, j: (0, 0)),
        ],
        out_specs=pl.BlockSpec((1, 1, PT), lambda n, j: (n, 0, j)),
        out_shape=jax.ShapeDtypeStruct((N, 1, HW), jnp.int32),
        interpret=interpret,
    )(lat3, codebook)


def _sc_gather(idx_flat, codebook):
    mesh = plsc.VectorSubcoreMesh(core_axis_name="c", subcore_axis_name="s")

    @functools.partial(
        pl.kernel,
        mesh=mesh,
        compiler_params=pltpu.CompilerParams(use_tc_tiling_on_sc=False),
        out_type=jax.ShapeDtypeStruct((P, D), jnp.float32),
        scratch_types=[
            pltpu.VMEM((ROWS_PER_W,), jnp.int32),
            pltpu.VMEM((ROWS_PER_W, D), jnp.float32),
            pltpu.SemaphoreType.DMA,
        ],
    )
    def gather_kernel(idx_hbm, cb_hbm, out_hbm, idx_v, rows_v, sem):
        wid = lax.axis_index("s") * SC_NC + lax.axis_index("c")
        base = wid * ROWS_PER_W
        pltpu.sync_copy(idx_hbm.at[pl.ds(base, ROWS_PER_W)], idx_v)
        pltpu.async_copy(cb_hbm.at[idx_v], rows_v, sem).wait()
        pltpu.sync_copy(rows_v, out_hbm.at[pl.ds(base, ROWS_PER_W)])

    return gather_kernel(idx_flat, codebook)


def _loss_body(lat_ref, q_ref, tgt_ref, w_ref, b_ref,
               loss_ref, vq_ref, rec_ref):
    eye = (lax.broadcasted_iota(jnp.int32, (D, D), 0)
           == lax.broadcasted_iota(jnp.int32, (D, D), 1))
    w = w_ref[...]
    b = b_ref[...]
    cb_sum = jnp.float32(0.0)
    rec_sum = jnp.float32(0.0)
    for n in range(N):
        latn = lat_ref[n]
        qn = q_ref[pl.ds(n * HW, HW), :]
        m = lax.dot_general(latn, qn, (((1,), (0,)), ((), ())),
                            preferred_element_type=jnp.float32,
                            precision=lax.Precision.HIGHEST)
        cross = jnp.sum(jnp.where(eye, m, 0.0))
        cb_sum = cb_sum + jnp.sum(latn * latn) + jnp.sum(qn * qn) - 2.0 * cross
        out = lax.dot_general(w, qn, (((1,), (1,)), ((), ())),
                              preferred_element_type=jnp.float32,
                              precision=lax.Precision.HIGHEST) + b
        r = out - tgt_ref[n]
        rec_sum = rec_sum + jnp.sum(r * r)
    cb_mean = cb_sum / jnp.float32(P * D)
    vq = (VQLOSS_WEIGHT + VQLOSS_WEIGHT * COMMITMENT_WEIGHT) * cb_mean
    rec = rec_sum / jnp.float32(P * C_IMG)
    loss_ref[0, 0] = vq + rec
    vq_ref[0, 0] = vq
    rec_ref[0, 0] = rec


def _losses(lat3, q, tgt3, w_dec, b_dec2, interpret=False):
    smem_out = pl.BlockSpec(memory_space=pltpu.SMEM)
    return pl.pallas_call(
        _loss_body,
        in_specs=[pl.BlockSpec(memory_space=pltpu.VMEM)] * 5,
        out_specs=(smem_out, smem_out, smem_out),
        out_shape=(
            jax.ShapeDtypeStruct((1, 1), jnp.float32),
            jax.ShapeDtypeStruct((1, 1), jnp.float32),
            jax.ShapeDtypeStruct((1, 1), jnp.float32),
        ),
        interpret=interpret,
    )(lat3, q, tgt3, w_dec, b_dec2)


def kernel(latents, target, codebook, W_dec, b_dec):
    lat3 = latents.reshape(N, D, HW)
    tgt3 = target.reshape(N, C_IMG, HW)
    idx = _nearest_indices(lat3, codebook)
    q = _sc_gather(idx.reshape(P), codebook)
    loss, vq, rec = _losses(lat3, q, tgt3, W_dec, b_dec.reshape(C_IMG, 1))
    return (loss.reshape(()), vq.reshape(()), rec.reshape(()))


# speedup vs baseline: 4.8931x; 1.0104x over previous
import functools

import jax
import jax.numpy as jnp
from jax import lax
from jax.experimental import pallas as pl
from jax.experimental.pallas import tpu as pltpu
from jax.experimental.pallas import tpu_sc as plsc

N, D, HW = 8, 32, 1024
C_IMG = 3
K = 8192
PT = 1024
P = N * HW
COMMITMENT_WEIGHT = 0.25
VQLOSS_WEIGHT = 1.0

SC_NC, SC_NS = 2, 16
SC_NW = SC_NC * SC_NS
ROWS_PER_W = P // SC_NW


def _argmin_body(lat_ref, cb_ref, idx_ref):
    latp = lat_ref[0]
    cb = cb_ref[...]
    c2 = jnp.sum(cb * cb, axis=1, keepdims=True)
    bh = (-0.5 * c2).astype(jnp.bfloat16).astype(jnp.float32)
    bl = -0.5 * c2 - bh
    cb_aug = jnp.concatenate([cb, bh, bl], axis=1)
    lat_aug = jnp.concatenate(
        [latp, jnp.ones((2, PT), jnp.float32)], axis=0)
    s = lax.dot_general(cb_aug, lat_aug, (((1,), (0,)), ((), ())),
                        preferred_element_type=jnp.float32,
                        precision=lax.Precision.DEFAULT)
    idx_ref[0] = jnp.argmax(s, axis=0).astype(jnp.int32)[None, :]


def _nearest_indices(lat3, codebook, interpret=False):
    return pl.pallas_call(
        _argmin_body,
        grid=(N, HW // PT),
        in_specs=[
            pl.BlockSpec((1, D, PT), lambda n, j: (n, 0, j)),
            pl.BlockSpec((K, D), lambda n, j: (0, 0)),
        ],
        out_specs=pl.BlockSpec((1, 1, PT), lambda n, j: (n, 0, j)),
        out_shape=jax.ShapeDtypeStruct((N, 1, HW), jnp.int32),
        interpret=interpret,
    )(lat3, codebook)


def _sc_gather(idx_flat, codebook):
    mesh = plsc.VectorSubcoreMesh(core_axis_name="c", subcore_axis_name="s")

    @functools.partial(
        pl.kernel,
        mesh=mesh,
        compiler_params=pltpu.CompilerParams(use_tc_tiling_on_sc=False),
        out_type=jax.ShapeDtypeStruct((P, D), jnp.float32),
        scratch_types=[
            pltpu.VMEM((ROWS_PER_W,), jnp.int32),
            pltpu.VMEM((ROWS_PER_W, D), jnp.float32),
            pltpu.SemaphoreType.DMA,
        ],
    )
    def gather_kernel(idx_hbm, cb_hbm, out_hbm, idx_v, rows_v, sem):
        wid = lax.axis_index("s") * SC_NC + lax.axis_index("c")
        base = wid * ROWS_PER_W
        pltpu.sync_copy(idx_hbm.at[pl.ds(base, ROWS_PER_W)], idx_v)
        pltpu.async_copy(cb_hbm.at[idx_v], rows_v, sem).wait()
        pltpu.sync_copy(rows_v, out_hbm.at[pl.ds(base, ROWS_PER_W)])

    return gather_kernel(idx_flat, codebook)


def _loss_body(lat_ref, q_ref, tgt_ref, w_ref, b_ref,
               loss_ref, vq_ref, rec_ref):
    eye = (lax.broadcasted_iota(jnp.int32, (D, D), 0)
           == lax.broadcasted_iota(jnp.int32, (D, D), 1))
    w = w_ref[...]
    b = b_ref[...]
    cb_sum = jnp.float32(0.0)
    rec_sum = jnp.float32(0.0)
    for n in range(N):
        latn = lat_ref[n]
        qn = q_ref[pl.ds(n * HW, HW), :]
        m = lax.dot_general(latn, qn, (((1,), (0,)), ((), ())),
                            preferred_element_type=jnp.float32,
                            precision=lax.Precision.HIGHEST)
        cross = jnp.sum(jnp.where(eye, m, 0.0))
        cb_sum = cb_sum + jnp.sum(latn * latn) + jnp.sum(qn * qn) - 2.0 * cross
        out = lax.dot_general(w, qn, (((1,), (1,)), ((), ())),
                              preferred_element_type=jnp.float32,
                              precision=lax.Precision.HIGHEST) + b
        r = out - tgt_ref[n]
        rec_sum = rec_sum + jnp.sum(r * r)
    cb_mean = cb_sum / jnp.float32(P * D)
    vq = (VQLOSS_WEIGHT + VQLOSS_WEIGHT * COMMITMENT_WEIGHT) * cb_mean
    rec = rec_sum / jnp.float32(P * C_IMG)
    loss_ref[0, 0] = vq + rec
    vq_ref[0, 0] = vq
    rec_ref[0, 0] = rec


def _losses(lat3, q, tgt3, w_dec, b_dec2, interpret=False):
    smem_out = pl.BlockSpec(memory_space=pltpu.SMEM)
    return pl.pallas_call(
        _loss_body,
        in_specs=[pl.BlockSpec(memory_space=pltpu.VMEM)] * 5,
        out_specs=(smem_out, smem_out, smem_out),
        out_shape=(
            jax.ShapeDtypeStruct((1, 1), jnp.float32),
            jax.ShapeDtypeStruct((1, 1), jnp.float32),
            jax.ShapeDtypeStruct((1, 1), jnp.float32),
        ),
        interpret=interpret,
    )(lat3, q, tgt3, w_dec, b_dec2)


def kernel(latents, target, codebook, W_dec, b_dec):
    lat3 = latents.reshape(N, D, HW)
    tgt3 = target.reshape(N, C_IMG, HW)
    idx = _nearest_indices(lat3, codebook)
    q = _sc_gather(idx.reshape(P), codebook)
    loss, vq, rec = _losses(lat3, q, tgt3, W_dec, b_dec.reshape(C_IMG, 1))
    return (loss.reshape(()), vq.reshape(()), rec.reshape(()))


# speedup vs baseline: 4.8984x; 1.0011x over previous
import functools

import jax
import jax.numpy as jnp
from jax import lax
from jax.experimental import pallas as pl
from jax.experimental.pallas import tpu as pltpu
from jax.experimental.pallas import tpu_sc as plsc

N, D, HW = 8, 32, 1024
C_IMG = 3
K = 8192
PT = 1024
P = N * HW
COMMITMENT_WEIGHT = 0.25
VQLOSS_WEIGHT = 1.0

SC_NC, SC_NS = 2, 16
SC_NW = SC_NC * SC_NS
ROWS_PER_W = P // SC_NW


def _argmin_body(lat_ref, cb_ref, idx_ref):
    latp = lat_ref[0]
    cb = cb_ref[...]
    c2 = jnp.sum(cb * cb, axis=1, keepdims=True)
    bh = (-0.5 * c2).astype(jnp.bfloat16).astype(jnp.float32)
    bl = -0.5 * c2 - bh
    cb_aug = jnp.concatenate([cb, bh, bl], axis=1)
    lat_aug = jnp.concatenate(
        [latp, jnp.ones((2, PT), jnp.float32)], axis=0)
    HPT = PT // 2
    for j in range(2):
        s = lax.dot_general(cb_aug, lat_aug[:, j * HPT:(j + 1) * HPT],
                            (((1,), (0,)), ((), ())),
                            preferred_element_type=jnp.float32,
                            precision=lax.Precision.DEFAULT)
        idx_ref[0, 0, pl.ds(j * HPT, HPT)] = jnp.argmax(s, axis=0).astype(jnp.int32)


def _nearest_indices(lat3, codebook, interpret=False):
    return pl.pallas_call(
        _argmin_body,
        grid=(N, HW // PT),
        in_specs=[
            pl.BlockSpec((1, D, PT), lambda n, j: (n, 0, j)),
            pl.BlockSpec((K, D), lambda n, j: (0, 0)),
        ],
        out_specs=pl.BlockSpec((1, 1, PT), lambda n, j: (n, 0, j)),
        out_shape=jax.ShapeDtypeStruct((N, 1, HW), jnp.int32),
        interpret=interpret,
    )(lat3, codebook)


def _sc_gather(idx_flat, codebook):
    mesh = plsc.VectorSubcoreMesh(core_axis_name="c", subcore_axis_name="s")

    @functools.partial(
        pl.kernel,
        mesh=mesh,
        compiler_params=pltpu.CompilerParams(use_tc_tiling_on_sc=False),
        out_type=jax.ShapeDtypeStruct((P, D), jnp.float32),
        scratch_types=[
            pltpu.VMEM((ROWS_PER_W,), jnp.int32),
            pltpu.VMEM((ROWS_PER_W, D), jnp.float32),
            pltpu.SemaphoreType.DMA,
        ],
    )
    def gather_kernel(idx_hbm, cb_hbm, out_hbm, idx_v, rows_v, sem):
        wid = lax.axis_index("s") * SC_NC + lax.axis_index("c")
        base = wid * ROWS_PER_W
        pltpu.sync_copy(idx_hbm.at[pl.ds(base, ROWS_PER_W)], idx_v)
        pltpu.async_copy(cb_hbm.at[idx_v], rows_v, sem).wait()
        pltpu.sync_copy(rows_v, out_hbm.at[pl.ds(base, ROWS_PER_W)])

    return gather_kernel(idx_flat, codebook)


def _loss_body(lat_ref, q_ref, tgt_ref, w_ref, b_ref,
               loss_ref, vq_ref, rec_ref):
    eye = (lax.broadcasted_iota(jnp.int32, (D, D), 0)
           == lax.broadcasted_iota(jnp.int32, (D, D), 1))
    w = w_ref[...]
    b = b_ref[...]
    cb_sum = jnp.float32(0.0)
    rec_sum = jnp.float32(0.0)
    for n in range(N):
        latn = lat_ref[n]
        qn = q_ref[pl.ds(n * HW, HW), :]
        m = lax.dot_general(latn, qn, (((1,), (0,)), ((), ())),
                            preferred_element_type=jnp.float32,
                            precision=lax.Precision.HIGHEST)
        cross = jnp.sum(jnp.where(eye, m, 0.0))
        cb_sum = cb_sum + jnp.sum(latn * latn) + jnp.sum(qn * qn) - 2.0 * cross
        out = lax.dot_general(w, qn, (((1,), (1,)), ((), ())),
                              preferred_element_type=jnp.float32,
                              precision=lax.Precision.HIGHEST) + b
        r = out - tgt_ref[n]
        rec_sum = rec_sum + jnp.sum(r * r)
    cb_mean = cb_sum / jnp.float32(P * D)
    vq = (VQLOSS_WEIGHT + VQLOSS_WEIGHT * COMMITMENT_WEIGHT) * cb_mean
    rec = rec_sum / jnp.float32(P * C_IMG)
    loss_ref[0, 0] = vq + rec
    vq_ref[0, 0] = vq
    rec_ref[0, 0] = rec


def _losses(lat3, q, tgt3, w_dec, b_dec2, interpret=False):
    smem_out = pl.BlockSpec(memory_space=pltpu.SMEM)
    return pl.pallas_call(
        _loss_body,
        in_specs=[pl.BlockSpec(memory_space=pltpu.VMEM)] * 5,
        out_specs=(smem_out, smem_out, smem_out),
        out_shape=(
            jax.ShapeDtypeStruct((1, 1), jnp.float32),
            jax.ShapeDtypeStruct((1, 1), jnp.float32),
            jax.ShapeDtypeStruct((1, 1), jnp.float32),
        ),
        interpret=interpret,
    )(lat3, q, tgt3, w_dec, b_dec2)


def kernel(latents, target, codebook, W_dec, b_dec):
    lat3 = latents.reshape(N, D, HW)
    tgt3 = target.reshape(N, C_IMG, HW)
    idx = _nearest_indices(lat3, codebook)
    q = _sc_gather(idx.reshape(P), codebook)
    loss, vq, rec = _losses(lat3, q, tgt3, W_dec, b_dec.reshape(C_IMG, 1))
    return (loss.reshape(()), vq.reshape(()), rec.reshape(()))


# speedup vs baseline: 5.1380x; 1.0489x over previous
import functools

import jax
import jax.numpy as jnp
from jax import lax
from jax.experimental import pallas as pl
from jax.experimental.pallas import tpu as pltpu
from jax.experimental.pallas import tpu_sc as plsc

N, D, HW = 8, 32, 1024
C_IMG = 3
K = 8192
PT = 1024
P = N * HW
COMMITMENT_WEIGHT = 0.25
VQLOSS_WEIGHT = 1.0

SC_NC, SC_NS = 2, 16
SC_NW = SC_NC * SC_NS
ROWS_PER_W = P // SC_NW


def _argmin_body(lat_ref, cb_ref, idx_ref):
    latp = lat_ref[0]
    cb = cb_ref[...]
    c2 = jnp.sum(cb * cb, axis=1, keepdims=True)
    bh = (-0.5 * c2).astype(jnp.bfloat16).astype(jnp.float32)
    bl = -0.5 * c2 - bh
    cb_aug = jnp.concatenate([cb, bh, bl], axis=1)
    lat_aug = jnp.concatenate(
        [latp, jnp.ones((2, PT), jnp.float32)], axis=0)
    HPT = PT // 2
    for j in range(2):
        s = lax.dot_general(cb_aug, lat_aug[:, j * HPT:(j + 1) * HPT],
                            (((1,), (0,)), ((), ())),
                            preferred_element_type=jnp.float32,
                            precision=lax.Precision.DEFAULT)
        idx_ref[0, 0, pl.ds(j * HPT, HPT)] = jnp.argmax(s, axis=0).astype(jnp.int32)


def _nearest_indices(lat3, codebook, interpret=False):
    return pl.pallas_call(
        _argmin_body,
        grid=(N, HW // PT),
        in_specs=[
            pl.BlockSpec((1, D, PT), lambda n, j: (n, 0, j)),
            pl.BlockSpec((K, D), lambda n, j: (0, 0)),
        ],
        out_specs=pl.BlockSpec((1, 1, PT), lambda n, j: (n, 0, j)),
        out_shape=jax.ShapeDtypeStruct((N, 1, HW), jnp.int32),
        interpret=interpret,
    )(lat3, codebook)


def _sc_gather(idx_flat, codebook):
    mesh = plsc.VectorSubcoreMesh(core_axis_name="c", subcore_axis_name="s")

    @functools.partial(
        pl.kernel,
        mesh=mesh,
        compiler_params=pltpu.CompilerParams(use_tc_tiling_on_sc=False),
        out_type=jax.ShapeDtypeStruct((P, D), jnp.float32),
        scratch_types=[
            pltpu.VMEM((ROWS_PER_W,), jnp.int32),
            pltpu.VMEM((ROWS_PER_W, D), jnp.float32),
            pltpu.SemaphoreType.DMA,
        ],
    )
    def gather_kernel(idx_hbm, cb_hbm, out_hbm, idx_v, rows_v, sem):
        wid = lax.axis_index("s") * SC_NC + lax.axis_index("c")
        base = wid * ROWS_PER_W
        pltpu.sync_copy(idx_hbm.at[pl.ds(base, ROWS_PER_W)], idx_v)
        pltpu.async_copy(cb_hbm.at[idx_v], rows_v, sem).wait()
        pltpu.sync_copy(rows_v, out_hbm.at[pl.ds(base, ROWS_PER_W)])

    return gather_kernel(idx_flat, codebook)


def _loss_body(lat_ref, q_ref, tgt_ref, w_ref, b_ref,
               loss_ref, vq_ref, rec_ref):
    eye = (lax.broadcasted_iota(jnp.int32, (D, D), 0)
           == lax.broadcasted_iota(jnp.int32, (D, D), 1))
    w = w_ref[...]
    b = b_ref[...]
    cb_sum = jnp.float32(0.0)
    rec_sum = jnp.float32(0.0)
    for n in range(N):
        latn = lat_ref[n]
        qn = q_ref[pl.ds(n * HW, HW), :]
        m = lax.dot_general(latn, qn, (((1,), (0,)), ((), ())),
                            preferred_element_type=jnp.float32,
                            precision=lax.Precision.DEFAULT)
        cross = jnp.sum(jnp.where(eye, m, 0.0))
        cb_sum = cb_sum + jnp.sum(latn * latn) + jnp.sum(qn * qn) - 2.0 * cross
        out = lax.dot_general(w, qn, (((1,), (1,)), ((), ())),
                              preferred_element_type=jnp.float32,
                              precision=lax.Precision.DEFAULT) + b
        r = out - tgt_ref[n]
        rec_sum = rec_sum + jnp.sum(r * r)
    cb_mean = cb_sum / jnp.float32(P * D)
    vq = (VQLOSS_WEIGHT + VQLOSS_WEIGHT * COMMITMENT_WEIGHT) * cb_mean
    rec = rec_sum / jnp.float32(P * C_IMG)
    loss_ref[0, 0] = vq + rec
    vq_ref[0, 0] = vq
    rec_ref[0, 0] = rec


def _losses(lat3, q, tgt3, w_dec, b_dec2, interpret=False):
    smem_out = pl.BlockSpec(memory_space=pltpu.SMEM)
    return pl.pallas_call(
        _loss_body,
        in_specs=[pl.BlockSpec(memory_space=pltpu.VMEM)] * 5,
        out_specs=(smem_out, smem_out, smem_out),
        out_shape=(
            jax.ShapeDtypeStruct((1, 1), jnp.float32),
            jax.ShapeDtypeStruct((1, 1), jnp.float32),
            jax.ShapeDtypeStruct((1, 1), jnp.float32),
        ),
        interpret=interpret,
    )(lat3, q, tgt3, w_dec, b_dec2)


def kernel(latents, target, codebook, W_dec, b_dec):
    lat3 = latents.reshape(N, D, HW)
    tgt3 = target.reshape(N, C_IMG, HW)
    idx = _nearest_indices(lat3, codebook)
    q = _sc_gather(idx.reshape(P), codebook)
    loss, vq, rec = _losses(lat3, q, tgt3, W_dec, b_dec.reshape(C_IMG, 1))
    return (loss.reshape(()), vq.reshape(()), rec.reshape(()))


# speedup vs baseline: 5.1593x; 1.0041x over previous
import functools

import jax
import jax.numpy as jnp
from jax import lax
from jax.experimental import pallas as pl
from jax.experimental.pallas import tpu as pltpu
from jax.experimental.pallas import tpu_sc as plsc

N, D, HW = 8, 32, 1024
C_IMG = 3
K = 8192
PT = 1024
P = N * HW
COMMITMENT_WEIGHT = 0.25
VQLOSS_WEIGHT = 1.0

SC_NC, SC_NS = 2, 16
SC_NW = SC_NC * SC_NS
ROWS_PER_W = P // SC_NW


def _argmin_body(lat_ref, cb_ref, idx_ref):
    latp = lat_ref[0]
    cb = cb_ref[...]
    c2 = jnp.sum(cb * cb, axis=1, keepdims=True)
    bh = (-0.5 * c2).astype(jnp.bfloat16).astype(jnp.float32)
    bl = -0.5 * c2 - bh
    cb_aug = jnp.concatenate([cb, bh, bl], axis=1)
    lat_aug = jnp.concatenate(
        [latp, jnp.ones((2, PT), jnp.float32)], axis=0)
    HPT = PT // 2
    for j in range(2):
        s = lax.dot_general(cb_aug, lat_aug[:, j * HPT:(j + 1) * HPT],
                            (((1,), (0,)), ((), ())),
                            preferred_element_type=jnp.float32,
                            precision=lax.Precision.DEFAULT)
        idx_ref[0, 0, pl.ds(j * HPT, HPT)] = jnp.argmax(s, axis=0).astype(jnp.int32)


def _nearest_indices(lat3, codebook):
    return pl.pallas_call(
        _argmin_body,
        grid=(N, HW // PT),
        in_specs=[
            pl.BlockSpec((1, D, PT), lambda n, j: (n, 0, j)),
            pl.BlockSpec((K, D), lambda n, j: (0, 0)),
        ],
        out_specs=pl.BlockSpec((1, 1, PT), lambda n, j: (n, 0, j)),
        out_shape=jax.ShapeDtypeStruct((N, 1, HW), jnp.int32),
    )(lat3, codebook)


def _sc_gather(idx_flat, codebook):
    mesh = plsc.VectorSubcoreMesh(core_axis_name="c", subcore_axis_name="s")

    @functools.partial(
        pl.kernel,
        mesh=mesh,
        compiler_params=pltpu.CompilerParams(use_tc_tiling_on_sc=False),
        out_type=jax.ShapeDtypeStruct((P, D), jnp.float32),
        scratch_types=[
            pltpu.VMEM((ROWS_PER_W,), jnp.int32),
            pltpu.VMEM((ROWS_PER_W, D), jnp.float32),
            pltpu.SemaphoreType.DMA,
        ],
    )
    def gather_kernel(idx_hbm, cb_hbm, out_hbm, idx_v, rows_v, sem):
        wid = lax.axis_index("s") * SC_NC + lax.axis_index("c")
        base = wid * ROWS_PER_W
        pltpu.sync_copy(idx_hbm.at[pl.ds(base, ROWS_PER_W)], idx_v)
        pltpu.async_copy(cb_hbm.at[idx_v], rows_v, sem).wait()
        pltpu.sync_copy(rows_v, out_hbm.at[pl.ds(base, ROWS_PER_W)])

    return gather_kernel(idx_flat, codebook)


def _loss_body(lat_ref, q_ref, tgt_ref, w_ref, b_ref,
               loss_ref, vq_ref, rec_ref):
    eye = (lax.broadcasted_iota(jnp.int32, (D, D), 0)
           == lax.broadcasted_iota(jnp.int32, (D, D), 1))
    w = w_ref[...]
    b = b_ref[...]
    cb_sum = jnp.float32(0.0)
    rec_sum = jnp.float32(0.0)
    for n in range(N):
        latn = lat_ref[n]
        qn = q_ref[pl.ds(n * HW, HW), :]
        m = lax.dot_general(latn, qn, (((1,), (0,)), ((), ())),
                            preferred_element_type=jnp.float32,
                            precision=lax.Precision.DEFAULT)
        cross = jnp.sum(jnp.where(eye, m, 0.0))
        cb_sum = cb_sum + jnp.sum(latn * latn) + jnp.sum(qn * qn) - 2.0 * cross
        out = lax.dot_general(w, qn, (((1,), (1,)), ((), ())),
                              preferred_element_type=jnp.float32,
                              precision=lax.Precision.DEFAULT) + b
        r = out - tgt_ref[n]
        rec_sum = rec_sum + jnp.sum(r * r)
    cb_mean = cb_sum / jnp.float32(P * D)
    vq = (VQLOSS_WEIGHT + VQLOSS_WEIGHT * COMMITMENT_WEIGHT) * cb_mean
    rec = rec_sum / jnp.float32(P * C_IMG)
    loss_ref[0, 0] = vq + rec
    vq_ref[0, 0] = vq
    rec_ref[0, 0] = rec


def _losses(lat3, q, tgt3, w_dec, b_dec2):
    smem_out = pl.BlockSpec(memory_space=pltpu.SMEM)
    return pl.pallas_call(
        _loss_body,
        in_specs=[pl.BlockSpec(memory_space=pltpu.VMEM)] * 5,
        out_specs=(smem_out, smem_out, smem_out),
        out_shape=(
            jax.ShapeDtypeStruct((1, 1), jnp.float32),
            jax.ShapeDtypeStruct((1, 1), jnp.float32),
            jax.ShapeDtypeStruct((1, 1), jnp.float32),
        ),
    )(lat3, q, tgt3, w_dec, b_dec2)


def kernel(latents, target, codebook, W_dec, b_dec):
    lat3 = latents.reshape(N, D, HW)
    tgt3 = target.reshape(N, C_IMG, HW)
    idx = _nearest_indices(lat3, codebook)
    q = _sc_gather(idx.reshape(P), codebook)
    loss, vq, rec = _losses(lat3, q, tgt3, W_dec, b_dec.reshape(C_IMG, 1))
    return (loss.reshape(()), vq.reshape(()), rec.reshape(()))
